# baseline (speedup 1.0000x reference)
import functools

import jax
import jax.numpy as jnp
from jax import lax
from jax.experimental import pallas as pl
from jax.experimental.pallas import tpu as pltpu
from jax.experimental.pallas import tpu_sc as plsc

WORD_NUM = 100000
EMBED = 128
BATCH = 16384
Y_MAX = 100.0
ALPHA = 0.75

_NC = 2
_NS = 16
_NW = _NC * _NS
_BPW = BATCH // _NW
_CH = 128
_NCHUNK = _BPW // _CH
_NJOB = 8
_NBUF = 7


def _sc_gather(ci3, ti3, ctx_tab, tgt_tab, ctx_bias, tgt_bias):
    mesh = plsc.VectorSubcoreMesh(core_axis_name="c", subcore_axis_name="s")
    f32 = jnp.float32
    rows_t = jax.ShapeDtypeStruct((_NW, _NCHUNK, _CH, EMBED), f32)
    bias_t = jax.ShapeDtypeStruct((_NW, _NCHUNK, _CH), f32)

    @functools.partial(
        pl.kernel,
        mesh=mesh,
        out_type=(rows_t, rows_t, bias_t, bias_t),
        scratch_types=[
            pltpu.VMEM((_NCHUNK, _CH), jnp.int32),
            pltpu.VMEM((_NCHUNK, _CH), jnp.int32),
            pltpu.VMEM((_NBUF, _CH, EMBED), f32),
            pltpu.VMEM((2, _NCHUNK, _CH), f32),
            pltpu.SemaphoreType.DMA,
            pltpu.SemaphoreType.DMA,
            pltpu.SemaphoreType.DMA,
        ],
    )
    def k(ci_hbm, ti_hbm, ct_hbm, tt_hbm, cb_hbm, tb_hbm,
          wc_out, wt_out, bc_out, bt_out, cidx_v, tidx_v, buf_v,
          bias_v, semg, semw, semb):
        wid = lax.axis_index("s") * _NC + lax.axis_index("c")
        tabs = (ct_hbm, tt_hbm)
        btabs = (cb_hbm, tb_hbm)
        routs = (wc_out, wt_out)
        bouts = (bc_out, bt_out)
        idxs = (cidx_v, tidx_v)

        pltpu.sync_copy(ci_hbm.at[wid], cidx_v)
        pltpu.sync_copy(ti_hbm.at[wid], tidx_v)

        bias_gets = []
        for t in range(2):
            for c in range(_NCHUNK):
                op = pltpu.make_async_copy(
                    btabs[t].at[idxs[t].at[c]], bias_v.at[t, c], semb)
                op.start()
                bias_gets.append(op)

        gath = {}

        def fire_job(j):
            t, c = j // _NCHUNK, j % _NCHUNK
            op = pltpu.make_async_copy(
                tabs[t].at[idxs[t].at[c]],
                buf_v.at[j % _NBUF], semg)
            op.start()
            gath[j] = op

        for j in range(_NBUF):
            fire_job(j)

        writes = {}
        for j in range(_NJOB):
            gath[j].wait()
            t, c = j // _NCHUNK, j % _NCHUNK
            wop = pltpu.make_async_copy(
                buf_v.at[j % _NBUF], routs[t].at[wid, c], semw)
            wop.start()
            writes[j] = wop
            if j + _NBUF < _NJOB:
                writes[j].wait()
                fire_job(j + _NBUF)

        for op in bias_gets:
            op.wait()
        bias_ws = []
        for t in range(2):
            wop = pltpu.make_async_copy(bias_v.at[t], bouts[t].at[wid], semb)
            wop.start()
            bias_ws.append(wop)
        for j in range(_NJOB):
            if j + _NBUF >= _NJOB:
                writes[j].wait()
        for wop in bias_ws:
            wop.wait()

    return k(ci3, ti3, ctx_tab, tgt_tab, ctx_bias, tgt_bias)


_BLK = 8192
_NB = BATCH // _BLK


def _tc_prep(y):
    def body(y_ref, w_ref, ly_ref):
        yv = y_ref[...]
        w_ref[...] = jnp.where(yv < Y_MAX, (yv / Y_MAX) ** ALPHA, 1.0)
        ly_ref[...] = jnp.log(yv)

    return pl.pallas_call(
        body,
        out_shape=(jax.ShapeDtypeStruct((BATCH,), jnp.float32),
                   jax.ShapeDtypeStruct((BATCH,), jnp.float32)),
    )(y)


_GRP = _BLK // EMBED


def _tc_loss(wc, wt, bc, bt, w, ly):
    def body(wc_ref, wt_ref, bc_ref, bt_ref, w_ref, ly_ref, out_ref,
             m_ref, f_ref):
        i = pl.program_id(0)

        @pl.when(i == 0)
        def _():
            bi = jax.lax.broadcasted_iota(jnp.int32, (_BLK, EMBED), 0)
            li = jax.lax.broadcasted_iota(jnp.int32, (_BLK, EMBED), 1)
            m_ref[...] = jnp.where((bi % EMBED) == li, 1.0, 0.0)
            gi = jax.lax.broadcasted_iota(jnp.int32, (_GRP, _BLK), 0)
            bj = jax.lax.broadcasted_iota(jnp.int32, (_GRP, _BLK), 1)
            f_ref[...] = jnp.where((bj // EMBED) == gi, 1.0, 0.0)
            out_ref[0, 0] = 0.0

        prod = (wc_ref[...] * wt_ref[...]).astype(jnp.bfloat16)
        ones_m = jnp.ones((EMBED, EMBED), jnp.bfloat16)
        dotm = jax.lax.dot_general(prod, ones_m, (((1,), (0,)), ((), ())),
                                   preferred_element_type=jnp.float32)
        z = dotm * m_ref[...]
        t = jax.lax.dot_general(f_ref[...], z, (((1,), (0,)), ((), ())),
                                preferred_element_type=jnp.float32)
        pred = t + bc_ref[...] + bt_ref[...]
        part = jnp.sum(w_ref[...] * (pred - ly_ref[...]) ** 2)
        out_ref[0, 0] += part

    tile = pl.BlockSpec((_GRP, EMBED), lambda i: (i, 0))
    out = pl.pallas_call(
        body,
        grid=(_NB,),
        in_specs=[
            pl.BlockSpec((_BLK, EMBED), lambda i: (i, 0)),
            pl.BlockSpec((_BLK, EMBED), lambda i: (i, 0)),
            tile, tile, tile, tile,
        ],
        out_specs=pl.BlockSpec(memory_space=pltpu.SMEM),
        out_shape=jax.ShapeDtypeStruct((1, 1), jnp.float32),
        scratch_shapes=[pltpu.VMEM((_BLK, EMBED), jnp.float32),
                        pltpu.VMEM((_GRP, _BLK), jnp.float32)],
    )(wc, wt, bc, bt, w, ly)
    return out[0, 0] / BATCH


def kernel(context_idx, target_idx, y, context_table, target_table, context_bias, target_bias):
    ci = context_idx.astype(jnp.int32).reshape(_NW, _NCHUNK, _CH)
    ti = target_idx.astype(jnp.int32).reshape(_NW, _NCHUNK, _CH)
    w, ly = _tc_prep(y)
    wc4, wt4, bc3, bt3 = _sc_gather(ci, ti, context_table, target_table,
                                    context_bias, target_bias)
    wc = wc4.reshape(BATCH, EMBED)
    wt = wt4.reshape(BATCH, EMBED)
    bc = bc3.reshape(BATCH // EMBED, EMBED)
    bt = bt3.reshape(BATCH // EMBED, EMBED)
    w2 = w.reshape(BATCH // EMBED, EMBED)
    ly2 = ly.reshape(BATCH // EMBED, EMBED)
    return _tc_loss(wc, wt, bc, bt, w2, ly2)


# speedup vs baseline: 1.0178x; 1.0178x over previous
import functools

import jax
import jax.numpy as jnp
from jax import lax
from jax.experimental import pallas as pl
from jax.experimental.pallas import tpu as pltpu
from jax.experimental.pallas import tpu_sc as plsc

WORD_NUM = 100000
EMBED = 128
BATCH = 16384
Y_MAX = 100.0
ALPHA = 0.75

_NC = 2
_NS = 16
_NW = _NC * _NS
_BPW = BATCH // _NW
_CH = 128
_NCHUNK = _BPW // _CH
_NJOB = 8
_NBUF = 7


def _sc_gather(ci3, ti3, ctx_tab, tgt_tab, ctx_bias, tgt_bias):
    mesh = plsc.VectorSubcoreMesh(core_axis_name="c", subcore_axis_name="s")
    f32 = jnp.float32
    rows_t = jax.ShapeDtypeStruct((_NW, _NCHUNK, _CH, EMBED), f32)
    bias_t = jax.ShapeDtypeStruct((_NW, _NCHUNK, _CH), f32)

    @functools.partial(
        pl.kernel,
        mesh=mesh,
        out_type=(rows_t, rows_t, bias_t, bias_t),
        scratch_types=[
            pltpu.VMEM((_NCHUNK, _CH), jnp.int32),
            pltpu.VMEM((_NCHUNK, _CH), jnp.int32),
            pltpu.VMEM((_NBUF, _CH, EMBED), f32),
            pltpu.VMEM((2, _NCHUNK, _CH), f32),
            pltpu.SemaphoreType.DMA,
            pltpu.SemaphoreType.DMA,
            pltpu.SemaphoreType.DMA,
        ],
    )
    def k(ci_hbm, ti_hbm, ct_hbm, tt_hbm, cb_hbm, tb_hbm,
          wc_out, wt_out, bc_out, bt_out, cidx_v, tidx_v, buf_v,
          bias_v, semg, semw, semb):
        wid = lax.axis_index("s") * _NC + lax.axis_index("c")
        tabs = (ct_hbm, tt_hbm)
        btabs = (cb_hbm, tb_hbm)
        routs = (wc_out, wt_out)
        bouts = (bc_out, bt_out)
        idxs = (cidx_v, tidx_v)

        pltpu.sync_copy(ci_hbm.at[wid], cidx_v)
        pltpu.sync_copy(ti_hbm.at[wid], tidx_v)

        bias_gets = []
        for t in range(2):
            for c in range(_NCHUNK):
                op = pltpu.make_async_copy(
                    btabs[t].at[idxs[t].at[c]], bias_v.at[t, c], semb)
                op.start()
                bias_gets.append(op)

        gath = {}

        def fire_job(j):
            t, c = j // _NCHUNK, j % _NCHUNK
            op = pltpu.make_async_copy(
                tabs[t].at[idxs[t].at[c]],
                buf_v.at[j % _NBUF], semg)
            op.start()
            gath[j] = op

        for j in range(_NBUF):
            fire_job(j)

        writes = {}
        for j in range(_NJOB):
            gath[j].wait()
            t, c = j // _NCHUNK, j % _NCHUNK
            wop = pltpu.make_async_copy(
                buf_v.at[j % _NBUF], routs[t].at[wid, c], semw)
            wop.start()
            writes[j] = wop
            if j + _NBUF < _NJOB:
                writes[j].wait()
                fire_job(j + _NBUF)

        for op in bias_gets:
            op.wait()
        bias_ws = []
        for t in range(2):
            wop = pltpu.make_async_copy(bias_v.at[t], bouts[t].at[wid], semb)
            wop.start()
            bias_ws.append(wop)
        for j in range(_NJOB):
            if j + _NBUF >= _NJOB:
                writes[j].wait()
        for wop in bias_ws:
            wop.wait()

    return k(ci3, ti3, ctx_tab, tgt_tab, ctx_bias, tgt_bias)


_BLK = 8192
_NB = BATCH // _BLK


def _tc_prep(y):
    def body(y_ref, w_ref, ly_ref):
        yv = y_ref[...]
        w_ref[...] = jnp.where(yv < Y_MAX, (yv / Y_MAX) ** ALPHA, 1.0)
        ly_ref[...] = jnp.log(yv)

    return pl.pallas_call(
        body,
        out_shape=(jax.ShapeDtypeStruct((BATCH,), jnp.float32),
                   jax.ShapeDtypeStruct((BATCH,), jnp.float32)),
    )(y)


_GRP = _BLK // EMBED


def _tc_loss(wc, wt, bc, bt, w, ly):
    def body(wc_ref, wt_ref, bc_ref, bt_ref, w_ref, ly_ref, out_ref, m_ref):
        i = pl.program_id(0)

        @pl.when(i == 0)
        def _():
            bi = jax.lax.broadcasted_iota(jnp.int32, (_BLK, EMBED), 0)
            li = jax.lax.broadcasted_iota(jnp.int32, (_BLK, EMBED), 1)
            m_ref[...] = jnp.where((bi % EMBED) == li, 1.0, 0.0)
            out_ref[0, 0] = 0.0

        prod = (wc_ref[...] * wt_ref[...]).astype(jnp.bfloat16)
        ones_m = jnp.ones((EMBED, EMBED), jnp.bfloat16)
        dotm = jax.lax.dot_general(prod, ones_m, (((1,), (0,)), ((), ())),
                                   preferred_element_type=jnp.float32)
        z = dotm * m_ref[...]
        t = jnp.sum(z.reshape(_GRP, EMBED, EMBED), axis=1)
        pred = t + bc_ref[...] + bt_ref[...]
        part = jnp.sum(w_ref[...] * (pred - ly_ref[...]) ** 2)
        out_ref[0, 0] += part

    tile = pl.BlockSpec((_GRP, EMBED), lambda i: (i, 0))
    out = pl.pallas_call(
        body,
        grid=(_NB,),
        in_specs=[
            pl.BlockSpec((_BLK, EMBED), lambda i: (i, 0)),
            pl.BlockSpec((_BLK, EMBED), lambda i: (i, 0)),
            tile, tile, tile, tile,
        ],
        out_specs=pl.BlockSpec(memory_space=pltpu.SMEM),
        out_shape=jax.ShapeDtypeStruct((1, 1), jnp.float32),
        scratch_shapes=[pltpu.VMEM((_BLK, EMBED), jnp.float32)],
    )(wc, wt, bc, bt, w, ly)
    return out[0, 0] / BATCH


def kernel(context_idx, target_idx, y, context_table, target_table, context_bias, target_bias):
    ci = context_idx.astype(jnp.int32).reshape(_NW, _NCHUNK, _CH)
    ti = target_idx.astype(jnp.int32).reshape(_NW, _NCHUNK, _CH)
    w, ly = _tc_prep(y)
    wc4, wt4, bc3, bt3 = _sc_gather(ci, ti, context_table, target_table,
                                    context_bias, target_bias)
    wc = wc4.reshape(BATCH, EMBED)
    wt = wt4.reshape(BATCH, EMBED)
    bc = bc3.reshape(BATCH // EMBED, EMBED)
    bt = bt3.reshape(BATCH // EMBED, EMBED)
    w2 = w.reshape(BATCH // EMBED, EMBED)
    ly2 = ly.reshape(BATCH // EMBED, EMBED)
    return _tc_loss(wc, wt, bc, bt, w2, ly2)
